# baseline (speedup 1.0000x reference)
import functools

import jax
import jax.numpy as jnp
from jax import lax
from jax.experimental import pallas as pl
from jax.experimental.pallas import tpu as pltpu
from jax.experimental.pallas import tpu_sc as plsc

D_MODEL = 1024
N_ROWS = 8192
N_WORKERS = 32
N_SUBCORES = 16
ROWS_PER_WORKER = N_ROWS // N_WORKERS
C = 8
N_CHUNKS = ROWS_PER_WORKER // C
NBUF = 4
N_OUTER = N_CHUNKS // NBUF
LANES = 16
SLICES_PER_ROW = D_MODEL // LANES


@functools.partial(
    pl.kernel,
    mesh=plsc.VectorSubcoreMesh(core_axis_name="c", subcore_axis_name="s"),
    out_type=jax.ShapeDtypeStruct((N_ROWS, D_MODEL), jnp.float32),
    scratch_types=[
        pltpu.VMEM((ROWS_PER_WORKER,), jnp.int32),
        pltpu.VMEM((NBUF, C, D_MODEL), jnp.float32),
        pltpu.VMEM((NBUF, C, D_MODEL), jnp.float32),
        pltpu.VMEM_SHARED((N_SUBCORES, NBUF, C, D_MODEL), jnp.float32),
    ]
    + [pltpu.SemaphoreType.DMA] * (4 * NBUF),
)
def _pos_encode(x_hbm, idx_hbm, table_hbm, out_hbm, idx_v, xbufs, rowsbufs,
                xshared, *sems):
    sem_xs = sems[0:NBUF]
    sem_s2t = sems[NBUF:2 * NBUF]
    sem_g = sems[2 * NBUF:3 * NBUF]
    sem_out = sems[3 * NBUF:4 * NBUF]

    sid = lax.axis_index("s")
    wid = sid * 2 + lax.axis_index("c")
    row0 = wid * ROWS_PER_WORKER

    def start_xs(g, b):
        pltpu.async_copy(x_hbm.at[pl.ds(row0 + g * C, C)],
                         xshared.at[sid, b], sem_xs[b])

    def wait_xs(b):
        pltpu.make_async_copy(x_hbm.at[pl.ds(row0, C)], xshared.at[sid, b],
                              sem_xs[b]).wait()

    def start_s2t(b):
        pltpu.async_copy(xshared.at[sid, b], xbufs.at[b], sem_s2t[b])

    def wait_s2t(b):
        pltpu.make_async_copy(xshared.at[sid, b], xbufs.at[b],
                              sem_s2t[b]).wait()

    def start_gather(g, b):
        pltpu.async_copy(table_hbm.at[idx_v.at[pl.ds(g * C, C)]],
                         rowsbufs.at[b], sem_g[b])

    def start_store(g, b):
        pltpu.async_copy(rowsbufs.at[b], out_hbm.at[pl.ds(row0 + g * C, C)],
                         sem_out[b])

    def wait_g(b):
        pltpu.make_async_copy(table_hbm.at[pl.ds(0, C)], rowsbufs.at[b],
                              sem_g[b]).wait()

    def wait_store(b):
        pltpu.make_async_copy(rowsbufs.at[b], out_hbm.at[pl.ds(row0, C)],
                              sem_out[b]).wait()

    for b in range(NBUF):
        start_xs(b, b)
    pltpu.sync_copy(idx_hbm.at[pl.ds(row0, ROWS_PER_WORKER)], idx_v)
    start_gather(0, 0)
    start_gather(1, 1)
    wait_xs(0)
    start_s2t(0)

    def outer(g0, carry):
        for b in range(NBUF):
            g = g0 * NBUF + b
            bn = (b + 2) % NBUF
            if b >= 2:
                wait_store(bn)
                @pl.when(g + 2 < N_CHUNKS)
                def _():
                    start_gather(g + 2, bn)
            else:
                @pl.when(g0 >= 1)
                def _():
                    wait_store(bn)
                start_gather(g + 2, bn)
            b1 = (b + 1) % NBUF
            @pl.when(g + 1 < N_CHUNKS)
            def _():
                wait_xs(b1)
                start_s2t(b1)
            wait_s2t(b)
            wait_g(b)

            def add_body(i, c, b=b):
                for r in range(C):
                    for jj in range(4):
                        sl = pl.ds(i * 4 * LANES + jj * LANES, LANES)
                        plsc.addupdate(rowsbufs.at[b, r, sl], xbufs[b, r, sl])
                return c

            lax.fori_loop(0, SLICES_PER_ROW // 4, add_body, 0)
            @pl.when(g0 < N_OUTER - 1)
            def _():
                start_xs(g + NBUF, b)
            start_store(g, b)
        return carry

    lax.fori_loop(0, N_OUTER, outer, 0)
    wait_store(2)
    wait_store(3)


def kernel(x, positions, pos_table):
    x2 = x.reshape(N_ROWS, D_MODEL)
    idx = positions.reshape(N_ROWS).astype(jnp.int32)
    out = _pos_encode(x2, idx, pos_table)
    return out.reshape(x.shape)


# speedup vs baseline: 1.4411x; 1.4411x over previous
import functools

import jax
import jax.numpy as jnp
from jax import lax
from jax.experimental import pallas as pl
from jax.experimental.pallas import tpu as pltpu
from jax.experimental.pallas import tpu_sc as plsc

D_MODEL = 1024
N_ROWS = 8192
N_WORKERS = 32
N_SUBCORES = 16
ROWS_PER_WORKER = N_ROWS // N_WORKERS
C = 8
N_CHUNKS = ROWS_PER_WORKER // C
NBUF = 4
N_OUTER = N_CHUNKS // NBUF
LANES = 16
SLICES_PER_ROW = D_MODEL // LANES


@functools.partial(
    pl.kernel,
    mesh=plsc.VectorSubcoreMesh(core_axis_name="c", subcore_axis_name="s"),
    out_type=jax.ShapeDtypeStruct((N_ROWS, D_MODEL), jnp.float32),
    scratch_types=[
        pltpu.VMEM((ROWS_PER_WORKER,), jnp.int32),
        pltpu.VMEM((NBUF, C, D_MODEL), jnp.float32),
        pltpu.VMEM((NBUF, C, D_MODEL), jnp.float32),
        pltpu.VMEM_SHARED((N_SUBCORES, NBUF, C, D_MODEL), jnp.float32),
    ]
    + [pltpu.SemaphoreType.DMA] * (4 * NBUF),
)
def _pos_encode(x_hbm, idx_hbm, table_hbm, out_hbm, idx_v, xbufs, rowsbufs,
                xshared, *sems):
    sem_xs = sems[0:NBUF]
    sem_s2t = sems[NBUF:2 * NBUF]
    sem_g = sems[2 * NBUF:3 * NBUF]
    sem_out = sems[3 * NBUF:4 * NBUF]

    sid = lax.axis_index("s")
    wid = sid * 2 + lax.axis_index("c")
    row0 = wid * ROWS_PER_WORKER

    def start_xs(g, b):
        pltpu.async_copy(x_hbm.at[pl.ds(row0 + g * C, C)],
                         xshared.at[sid, b], sem_xs[b])

    def wait_xs(b):
        pltpu.make_async_copy(x_hbm.at[pl.ds(row0, C)], xshared.at[sid, b],
                              sem_xs[b]).wait()

    def start_s2t(b):
        pltpu.async_copy(xshared.at[sid, b], xbufs.at[b], sem_s2t[b])

    def wait_s2t(b):
        pltpu.make_async_copy(xshared.at[sid, b], xbufs.at[b],
                              sem_s2t[b]).wait()

    def start_gather(g, b):
        pltpu.async_copy(table_hbm.at[idx_v.at[pl.ds(g * C, C)]],
                         rowsbufs.at[b], sem_g[b])

    def start_store(g, b):
        pltpu.async_copy(rowsbufs.at[b], out_hbm.at[pl.ds(row0 + g * C, C)],
                         sem_out[b])

    def wait_g(b):
        pltpu.make_async_copy(table_hbm.at[pl.ds(0, C)], rowsbufs.at[b],
                              sem_g[b]).wait()

    def wait_store(b):
        pltpu.make_async_copy(rowsbufs.at[b], out_hbm.at[pl.ds(row0, C)],
                              sem_out[b]).wait()

    for b in range(NBUF):
        start_xs(b, b)
    pltpu.sync_copy(idx_hbm.at[pl.ds(row0, ROWS_PER_WORKER)], idx_v)
    start_gather(0, 0)
    start_gather(1, 1)
    wait_xs(0)
    start_s2t(0)

    def outer(g0, carry):
        for b in range(NBUF):
            g = g0 * NBUF + b
            bn = (b + 2) % NBUF
            if b >= 2:
                wait_store(bn)
                @pl.when(g + 2 < N_CHUNKS)
                def _():
                    start_gather(g + 2, bn)
            else:
                @pl.when(g0 >= 1)
                def _():
                    wait_store(bn)
                start_gather(g + 2, bn)
            b1 = (b + 1) % NBUF
            @pl.when(g + 1 < N_CHUNKS)
            def _():
                wait_xs(b1)
                start_s2t(b1)
            wait_s2t(b)
            wait_g(b)

            @plsc.parallel_loop(0, SLICES_PER_ROW // 8)
            def add_body(i, b=b):
                for r in range(C):
                    for jj in range(8):
                        sl = pl.ds(i * 8 * LANES + jj * LANES, LANES)
                        plsc.addupdate(rowsbufs.at[b, r, sl], xbufs[b, r, sl])
            @pl.when(g0 < N_OUTER - 1)
            def _():
                start_xs(g + NBUF, b)
            start_store(g, b)
        return carry

    lax.fori_loop(0, N_OUTER, outer, 0)
    wait_store(2)
    wait_store(3)


def kernel(x, positions, pos_table):
    x2 = x.reshape(N_ROWS, D_MODEL)
    idx = positions.reshape(N_ROWS).astype(jnp.int32)
    out = _pos_encode(x2, idx, pos_table)
    return out.reshape(x.shape)
